# baseline (speedup 1.0000x reference)
import functools

import jax
import jax.numpy as jnp
from jax import lax
from jax.experimental import pallas as pl
from jax.experimental.pallas import tpu as pltpu
from jax.experimental.pallas import tpu_sc as plsc

N = 10000
E = 320000
IN_C = 128
OUT_C = 64
NUM_REL = 3
HALF = IN_C // 2
NSUB = 16
CHUNK = 32
NSLOT = 4
E_PAD = 321536
EPS = E_PAD // NSUB
NCHUNK = EPS // CHUNK
NBODY = NCHUNK // NSLOT
TBL = NUM_REL * N + 48
DUMMY_DST = 10008
CNT_PAD = 10240
CNT_PER_SUB = CNT_PAD // NSUB
CNT_Z = 80
NPS = 624
NPS_LAST = 640
ZROWS = 16
_SLICES = (0, 16)


def _sc_scatter(x2, ei3):
    mesh = plsc.VectorSubcoreMesh(core_axis_name="c", subcore_axis_name="s")

    @functools.partial(
        pl.kernel,
        mesh=mesh,
        compiler_params=pltpu.CompilerParams(use_tc_tiling_on_sc=False),
        out_type=[
            jax.ShapeDtypeStruct((NUM_REL, N, IN_C), jnp.float32),
            jax.ShapeDtypeStruct((2, CNT_PAD), jnp.float32),
        ],
        scratch_types=[
            pltpu.VMEM_SHARED((TBL, HALF), jnp.float32),
            pltpu.VMEM_SHARED((CNT_PAD,), jnp.float32),
        ] + [pltpu.VMEM((NUM_REL, CHUNK), jnp.int32)] * NSLOT
          + [pltpu.VMEM((CHUNK,), jnp.int32)] * NSLOT
          + [pltpu.VMEM((CHUNK,), jnp.int32)] * NSLOT
          + [pltpu.VMEM((CHUNK,), jnp.int32)] * NSLOT
          + [pltpu.VMEM((CHUNK, HALF), jnp.float32)] * NSLOT
          + [
            pltpu.VMEM((CHUNK,), jnp.float32),
            pltpu.VMEM((CNT_Z,), jnp.float32),
            pltpu.VMEM((ZROWS, HALF), jnp.float32),
        ] + [pltpu.SemaphoreType.DMA] * (4 * NSLOT + 1),
    )
    def k(x2_hbm, ei_hbm, s2_out, cnt_out,
          tbl, cnt_sp, ei0_v, ei1_v, ei2_v, ei3_v,
          comb0_v, comb1_v, comb2_v, comb3_v,
          srca0_v, srca1_v, srca2_v, srca3_v,
          dstc0_v, dstc1_v, dstc2_v, dstc3_v,
          rows0_v, rows1_v, rows2_v, rows3_v, ones_v, zc_v, zb_v,
          isem0, isem1, isem2, isem3, gsem0, gsem1, gsem2, gsem3,
          ssem0, ssem1, ssem2, ssem3, csem0, csem1, csem2, csem3, zsem):
        c = lax.axis_index("c")
        s = lax.axis_index("s")
        ei_b = (ei0_v, ei1_v, ei2_v, ei3_v)
        comb_b = (comb0_v, comb1_v, comb2_v, comb3_v)
        srca_b = (srca0_v, srca1_v, srca2_v, srca3_v)
        dstc_b = (dstc0_v, dstc1_v, dstc2_v, dstc3_v)
        rows_b = (rows0_v, rows1_v, rows2_v, rows3_v)
        isem_b = (isem0, isem1, isem2, isem3)
        gsem_b = (gsem0, gsem1, gsem2, gsem3)
        ssem_b = (ssem0, ssem1, ssem2, ssem3)
        csem_b = (csem0, csem1, csem2, csem3)

        zeros16 = jnp.zeros((16,), jnp.float32)
        ones16 = jnp.ones((16,), jnp.float32)

        def zc_body(i, _):
            zc_v[pl.ds(i * 16, 16)] = zeros16
            return 0
        lax.fori_loop(0, CNT_Z // 16, zc_body, 0)

        def zb_body(i, _):
            r = i // (HALF // 16)
            col = (i % (HALF // 16)) * 16
            zb_v[r, pl.ds(col, 16)] = zeros16
            return 0
        lax.fori_loop(0, ZROWS * HALF // 16, zb_body, 0)

        for st in _SLICES:
            ones_v[pl.ds(st, 16)] = ones16

        nb = s * NPS
        for r in range(NUM_REL):
            def zt_fire(i, _, r=r):
                pltpu.async_copy(zb_v, tbl.at[pl.ds(r * N + nb + i * ZROWS,
                                                    ZROWS)], zsem)
                return 0
            lax.fori_loop(0, NPS // ZROWS, zt_fire, 0)

            @pl.when(s == NSUB - 1)
            def _(r=r):
                pltpu.async_copy(zb_v.at[pl.ds(0, NPS_LAST - NPS)],
                                 tbl.at[pl.ds(r * N + NSUB * NPS,
                                              NPS_LAST - NPS)], zsem)
        for kk in range(CNT_PER_SUB // CNT_Z):
            pltpu.async_copy(zc_v, cnt_sp.at[pl.ds(s * CNT_PER_SUB + kk * CNT_Z,
                                                   CNT_Z)], zsem)
        for r in range(NUM_REL):
            def zt_wait(i, _, r=r):
                pltpu.make_async_copy(zb_v, tbl.at[pl.ds(r * N + nb + i * ZROWS,
                                                         ZROWS)], zsem).wait()
                return 0
            lax.fori_loop(0, NPS // ZROWS, zt_wait, 0)

            @pl.when(s == NSUB - 1)
            def _(r=r):
                pltpu.make_async_copy(zb_v.at[pl.ds(0, NPS_LAST - NPS)],
                                      tbl.at[pl.ds(r * N + NSUB * NPS,
                                                   NPS_LAST - NPS)],
                                      zsem).wait()
        for kk in range(CNT_PER_SUB // CNT_Z):
            pltpu.make_async_copy(zc_v,
                                  cnt_sp.at[pl.ds(s * CNT_PER_SUB + kk * CNT_Z,
                                                  CNT_Z)], zsem).wait()

        plsc.subcore_barrier()

        base_e = s * EPS

        def idx_fire(j, b):
            pltpu.async_copy(ei_hbm.at[:, pl.ds(base_e + j * CHUNK, CHUNK)],
                             ei_b[b], isem_b[b])

        def prep_and_gather(j, b):
            pltpu.make_async_copy(ei_hbm.at[:, pl.ds(base_e + j * CHUNK,
                                                     CHUNK)],
                                  ei_b[b], isem_b[b]).wait()
            for st in _SLICES:
                sl = pl.ds(st, 16)
                comb_b[b][sl] = ei_b[b][2, sl] * N + ei_b[b][1, sl]
                dstc_b[b][sl] = ei_b[b][1, sl]
                srca_b[b][sl] = ei_b[b][0, sl] * 2 + c

            @pl.when(j + NSLOT < NCHUNK)
            def _():
                idx_fire(j + NSLOT, b)
            pltpu.async_copy(x2_hbm.at[srca_b[b]], rows_b[b], gsem_b[b])

        def scatter(j, b):
            pltpu.make_async_copy(x2_hbm.at[srca_b[b]], rows_b[b],
                                  gsem_b[b]).wait()
            pltpu.async_copy(rows_b[b], tbl.at[comb_b[b]], ssem_b[b], add=True)

            @pl.when(c == (j % 2))
            def _():
                pltpu.async_copy(ones_v, cnt_sp.at[dstc_b[b]], csem_b[b],
                                 add=True)

        def drain_slot(b, par):
            pltpu.make_async_copy(rows_b[b], tbl.at[comb_b[b]],
                                  ssem_b[b]).wait()

            @pl.when(c == par)
            def _():
                pltpu.make_async_copy(ones_v, cnt_sp.at[dstc_b[b]],
                                      csem_b[b]).wait()

        for b in range(NSLOT):
            idx_fire(b, b)

        def ring_body(t, _):
            j0 = t * NSLOT
            for b in range(NSLOT):
                @pl.when(t > 0)
                def _(b=b):
                    drain_slot(b, (j0 - NSLOT + b) % 2)
                prep_and_gather(j0 + b, b)
            for b in range(NSLOT):
                scatter(j0 + b, b)
            return 0
        lax.fori_loop(0, NBODY, ring_body, 0)
        for b in range(NSLOT):
            drain_slot(b, (NCHUNK - NSLOT + b) % 2)

        plsc.subcore_barrier()

        for r in range(NUM_REL):
            @pl.when(s < NSUB - 1)
            def _(r=r):
                pltpu.async_copy(tbl.at[pl.ds(r * N + nb, NPS)],
                                 s2_out.at[r, pl.ds(nb, NPS),
                                           pl.ds(c * HALF, HALF)], zsem)

            @pl.when(s == NSUB - 1)
            def _(r=r):
                pltpu.async_copy(tbl.at[pl.ds(r * N + nb, NPS_LAST)],
                                 s2_out.at[r, pl.ds(nb, NPS_LAST),
                                           pl.ds(c * HALF, HALF)], zsem)

        pltpu.async_copy(cnt_sp.at[pl.ds(s * CNT_PER_SUB, CNT_PER_SUB)],
                         cnt_out.at[c, pl.ds(s * CNT_PER_SUB, CNT_PER_SUB)],
                         zsem)

        for r in range(NUM_REL):
            @pl.when(s < NSUB - 1)
            def _(r=r):
                pltpu.make_async_copy(tbl.at[pl.ds(r * N + nb, NPS)],
                                      s2_out.at[r, pl.ds(nb, NPS),
                                                pl.ds(c * HALF, HALF)],
                                      zsem).wait()

            @pl.when(s == NSUB - 1)
            def _(r=r):
                pltpu.make_async_copy(tbl.at[pl.ds(r * N + nb, NPS_LAST)],
                                      s2_out.at[r, pl.ds(nb, NPS_LAST),
                                                pl.ds(c * HALF, HALF)],
                                      zsem).wait()

        pltpu.make_async_copy(cnt_sp.at[pl.ds(s * CNT_PER_SUB, CNT_PER_SUB)],
                              cnt_out.at[c, pl.ds(s * CNT_PER_SUB,
                                                  CNT_PER_SUB)], zsem).wait()

    return k(x2, ei3)


def _tc_body(s2_ref, cnt_ref, x_ref, att_ref, basis_ref, root_ref,
             bi_ref, bc_ref, bo_ref, h_ref, c_ref):
    att = att_ref[...]
    recip = 1.0 / jnp.maximum(cnt_ref[...], 1.0)
    agg = jnp.zeros((s2_ref.shape[1], 3 * OUT_C), jnp.float32)
    for r in range(NUM_REL):
        w = (att[r, 0] * basis_ref[0]
             + att[r, 1] * basis_ref[1]
             + att[r, 2] * basis_ref[2])
        agg = agg + jnp.dot(s2_ref[r], w,
                            preferred_element_type=jnp.float32)
    g = agg * recip + jnp.dot(x_ref[...], root_ref[...],
                              preferred_element_type=jnp.float32)

    gi = jax.nn.sigmoid(g[:, 0:OUT_C] + bi_ref[...])
    gc = jnp.tanh(g[:, OUT_C:2 * OUT_C] + bc_ref[...])
    go = jax.nn.sigmoid(g[:, 2 * OUT_C:3 * OUT_C] + bo_ref[...])
    next_c = gi * gc
    c_ref[...] = next_c
    h_ref[...] = go * jnp.tanh(next_c)


def kernel(inputs, edge_index, edge_attr, basis_i, att_i, root_i,
           basis_h, att_h, root_h, bias_f, bias_i, bias_c, bias_o):
    del basis_h, att_h, root_h, bias_f
    x = inputs
    x2 = x.reshape(2 * N, HALF)
    ei3 = jnp.concatenate([edge_index.astype(jnp.int32),
                           edge_attr.astype(jnp.int32)[None, :]], axis=0)
    pad = jnp.broadcast_to(
        jnp.array([[0], [DUMMY_DST], [NUM_REL - 1]], jnp.int32),
        (3, E_PAD - E))
    ei3 = jnp.concatenate([ei3, pad], axis=1)

    s2, cnt = _sc_scatter(x2, ei3)

    cnt2 = (cnt[0, :N] + cnt[1, :N]).reshape(N, 1)
    basis2 = basis_i[:, :, OUT_C:]
    root_sl = root_i[:, OUT_C:]
    att_pad = jnp.zeros((8, 128), jnp.float32).at[:NUM_REL, :NUM_REL].set(att_i)
    b_i = bias_i.reshape(1, OUT_C)
    b_c = bias_c.reshape(1, OUT_C)
    b_o = bias_o.reshape(1, OUT_C)

    B = 2000
    grid = (N // B,)
    h, c = pl.pallas_call(
        _tc_body,
        grid=grid,
        in_specs=[
            pl.BlockSpec((NUM_REL, B, IN_C), lambda i: (0, i, 0)),
            pl.BlockSpec((B, 1), lambda i: (i, 0)),
            pl.BlockSpec((B, IN_C), lambda i: (i, 0)),
            pl.BlockSpec((8, 128), lambda i: (0, 0)),
            pl.BlockSpec((NUM_REL, IN_C, 3 * OUT_C), lambda i: (0, 0, 0)),
            pl.BlockSpec((IN_C, 3 * OUT_C), lambda i: (0, 0)),
            pl.BlockSpec((1, OUT_C), lambda i: (0, 0)),
            pl.BlockSpec((1, OUT_C), lambda i: (0, 0)),
            pl.BlockSpec((1, OUT_C), lambda i: (0, 0)),
        ],
        out_specs=[
            pl.BlockSpec((B, OUT_C), lambda i: (i, 0)),
            pl.BlockSpec((B, OUT_C), lambda i: (i, 0)),
        ],
        out_shape=[
            jax.ShapeDtypeStruct((N, OUT_C), jnp.float32),
            jax.ShapeDtypeStruct((N, OUT_C), jnp.float32),
        ],
    )(s2, cnt2, x, att_pad, basis2, root_sl, b_i, b_c, b_o)
    return (h, c)


# speedup vs baseline: 1.0829x; 1.0829x over previous
import functools

import jax
import jax.numpy as jnp
from jax import lax
from jax.experimental import pallas as pl
from jax.experimental.pallas import tpu as pltpu
from jax.experimental.pallas import tpu_sc as plsc

N = 10000
E = 320000
IN_C = 128
OUT_C = 64
NUM_REL = 3
HALF = IN_C // 2
NSUB = 16
CHUNK = 40
NSLOT = 3
E_PAD = 320640
EPS = E_PAD // NSUB
NCHUNK = EPS // CHUNK
NBODY = NCHUNK // NSLOT
TBL = NUM_REL * N + 48
DUMMY_DST = 10008
CNT_PAD = 10240
CNT_PER_SUB = CNT_PAD // NSUB
CNT_Z = 80
NPS = 624
NPS_LAST = 640
ZROWS = 16
_SLICES = (0, 16, 24)


def _sc_scatter(x2, ei3):
    mesh = plsc.VectorSubcoreMesh(core_axis_name="c", subcore_axis_name="s")

    @functools.partial(
        pl.kernel,
        mesh=mesh,
        compiler_params=pltpu.CompilerParams(use_tc_tiling_on_sc=False),
        out_type=[
            jax.ShapeDtypeStruct((NUM_REL, N, IN_C), jnp.float32),
            jax.ShapeDtypeStruct((2, CNT_PAD), jnp.float32),
        ],
        scratch_types=[
            pltpu.VMEM_SHARED((TBL, HALF), jnp.float32),
            pltpu.VMEM_SHARED((CNT_PAD,), jnp.float32),
        ] + [pltpu.VMEM((NUM_REL, CHUNK), jnp.int32)] * NSLOT
          + [pltpu.VMEM((CHUNK,), jnp.int32)] * NSLOT
          + [pltpu.VMEM((CHUNK,), jnp.int32)] * NSLOT
          + [pltpu.VMEM((CHUNK,), jnp.int32)] * NSLOT
          + [pltpu.VMEM((CHUNK, HALF), jnp.float32)] * NSLOT
          + [
            pltpu.VMEM((CHUNK,), jnp.float32),
            pltpu.VMEM((CNT_Z,), jnp.float32),
            pltpu.VMEM((ZROWS, HALF), jnp.float32),
        ] + [pltpu.SemaphoreType.DMA] * (4 * NSLOT + 1),
    )
    def k(x2_hbm, ei_hbm, s2_out, cnt_out,
          tbl, cnt_sp, ei0_v, ei1_v, ei2_v, comb0_v, comb1_v, comb2_v,
          srca0_v, srca1_v, srca2_v, dstc0_v, dstc1_v, dstc2_v,
          rows0_v, rows1_v, rows2_v, ones_v, zc_v, zb_v,
          isem0, isem1, isem2, gsem0, gsem1, gsem2,
          ssem0, ssem1, ssem2, csem0, csem1, csem2, zsem):
        c = lax.axis_index("c")
        s = lax.axis_index("s")
        ei_b = (ei0_v, ei1_v, ei2_v)
        comb_b = (comb0_v, comb1_v, comb2_v)
        srca_b = (srca0_v, srca1_v, srca2_v)
        dstc_b = (dstc0_v, dstc1_v, dstc2_v)
        rows_b = (rows0_v, rows1_v, rows2_v)
        isem_b = (isem0, isem1, isem2)
        gsem_b = (gsem0, gsem1, gsem2)
        ssem_b = (ssem0, ssem1, ssem2)
        csem_b = (csem0, csem1, csem2)

        zeros16 = jnp.zeros((16,), jnp.float32)
        ones16 = jnp.ones((16,), jnp.float32)

        def zc_body(i, _):
            zc_v[pl.ds(i * 16, 16)] = zeros16
            return 0
        lax.fori_loop(0, CNT_Z // 16, zc_body, 0)

        def zb_body(i, _):
            r = i // (HALF // 16)
            col = (i % (HALF // 16)) * 16
            zb_v[r, pl.ds(col, 16)] = zeros16
            return 0
        lax.fori_loop(0, ZROWS * HALF // 16, zb_body, 0)

        for st in _SLICES:
            ones_v[pl.ds(st, 16)] = ones16

        nb = s * NPS
        for r in range(NUM_REL):
            def zt_fire(i, _, r=r):
                pltpu.async_copy(zb_v, tbl.at[pl.ds(r * N + nb + i * ZROWS,
                                                    ZROWS)], zsem)
                return 0
            lax.fori_loop(0, NPS // ZROWS, zt_fire, 0)

            @pl.when(s == NSUB - 1)
            def _(r=r):
                pltpu.async_copy(zb_v.at[pl.ds(0, NPS_LAST - NPS)],
                                 tbl.at[pl.ds(r * N + NSUB * NPS,
                                              NPS_LAST - NPS)], zsem)
        for kk in range(CNT_PER_SUB // CNT_Z):
            pltpu.async_copy(zc_v, cnt_sp.at[pl.ds(s * CNT_PER_SUB + kk * CNT_Z,
                                                   CNT_Z)], zsem)
        for r in range(NUM_REL):
            def zt_wait(i, _, r=r):
                pltpu.make_async_copy(zb_v, tbl.at[pl.ds(r * N + nb + i * ZROWS,
                                                         ZROWS)], zsem).wait()
                return 0
            lax.fori_loop(0, NPS // ZROWS, zt_wait, 0)

            @pl.when(s == NSUB - 1)
            def _(r=r):
                pltpu.make_async_copy(zb_v.at[pl.ds(0, NPS_LAST - NPS)],
                                      tbl.at[pl.ds(r * N + NSUB * NPS,
                                                   NPS_LAST - NPS)],
                                      zsem).wait()
        for kk in range(CNT_PER_SUB // CNT_Z):
            pltpu.make_async_copy(zc_v,
                                  cnt_sp.at[pl.ds(s * CNT_PER_SUB + kk * CNT_Z,
                                                  CNT_Z)], zsem).wait()

        plsc.subcore_barrier()

        base_e = s * EPS

        def idx_fire(j, b):
            pltpu.async_copy(ei_hbm.at[:, pl.ds(base_e + j * CHUNK, CHUNK)],
                             ei_b[b], isem_b[b])

        def prep_and_gather(j, b):
            pltpu.make_async_copy(ei_hbm.at[:, pl.ds(base_e + j * CHUNK,
                                                     CHUNK)],
                                  ei_b[b], isem_b[b]).wait()
            for st in _SLICES:
                sl = pl.ds(st, 16)
                comb_b[b][sl] = ei_b[b][2, sl] * N + ei_b[b][1, sl]
                dstc_b[b][sl] = ei_b[b][1, sl]
                srca_b[b][sl] = ei_b[b][0, sl] * 2 + c

            @pl.when(j + NSLOT < NCHUNK)
            def _():
                idx_fire(j + NSLOT, b)
            pltpu.async_copy(x2_hbm.at[srca_b[b]], rows_b[b], gsem_b[b])

        def scatter(j, b):
            pltpu.make_async_copy(x2_hbm.at[srca_b[b]], rows_b[b],
                                  gsem_b[b]).wait()
            pltpu.async_copy(rows_b[b], tbl.at[comb_b[b]], ssem_b[b], add=True)

            @pl.when(c == (j % 2))
            def _():
                pltpu.async_copy(ones_v, cnt_sp.at[dstc_b[b]], csem_b[b],
                                 add=True)

        def drain_slot(b, par):
            pltpu.make_async_copy(rows_b[b], tbl.at[comb_b[b]],
                                  ssem_b[b]).wait()

            @pl.when(c == par)
            def _():
                pltpu.make_async_copy(ones_v, cnt_sp.at[dstc_b[b]],
                                      csem_b[b]).wait()

        for b in range(NSLOT):
            idx_fire(b, b)

        def ring_body(t, _):
            j0 = t * NSLOT
            for b in range(NSLOT):
                @pl.when(t > 0)
                def _(b=b):
                    drain_slot(b, (j0 - NSLOT + b) % 2)
                prep_and_gather(j0 + b, b)
            for b in range(NSLOT):
                scatter(j0 + b, b)
            return 0
        lax.fori_loop(0, NBODY, ring_body, 0)
        for b in range(NSLOT):
            drain_slot(b, (NCHUNK - NSLOT + b) % 2)

        plsc.subcore_barrier()

        for r in range(NUM_REL):
            @pl.when(s < NSUB - 1)
            def _(r=r):
                pltpu.async_copy(tbl.at[pl.ds(r * N + nb, NPS)],
                                 s2_out.at[r, pl.ds(nb, NPS),
                                           pl.ds(c * HALF, HALF)], zsem)

            @pl.when(s == NSUB - 1)
            def _(r=r):
                pltpu.async_copy(tbl.at[pl.ds(r * N + nb, NPS_LAST)],
                                 s2_out.at[r, pl.ds(nb, NPS_LAST),
                                           pl.ds(c * HALF, HALF)], zsem)

        pltpu.async_copy(cnt_sp.at[pl.ds(s * CNT_PER_SUB, CNT_PER_SUB)],
                         cnt_out.at[c, pl.ds(s * CNT_PER_SUB, CNT_PER_SUB)],
                         zsem)

        for r in range(NUM_REL):
            @pl.when(s < NSUB - 1)
            def _(r=r):
                pltpu.make_async_copy(tbl.at[pl.ds(r * N + nb, NPS)],
                                      s2_out.at[r, pl.ds(nb, NPS),
                                                pl.ds(c * HALF, HALF)],
                                      zsem).wait()

            @pl.when(s == NSUB - 1)
            def _(r=r):
                pltpu.make_async_copy(tbl.at[pl.ds(r * N + nb, NPS_LAST)],
                                      s2_out.at[r, pl.ds(nb, NPS_LAST),
                                                pl.ds(c * HALF, HALF)],
                                      zsem).wait()

        pltpu.make_async_copy(cnt_sp.at[pl.ds(s * CNT_PER_SUB, CNT_PER_SUB)],
                              cnt_out.at[c, pl.ds(s * CNT_PER_SUB,
                                                  CNT_PER_SUB)], zsem).wait()

    return k(x2, ei3)


def _tc_body(s2_ref, cnt_ref, x_ref, att_ref, basis_ref, root_ref,
             bi_ref, bc_ref, bo_ref, h_ref, c_ref):
    att = att_ref[...]
    recip = 1.0 / jnp.maximum(cnt_ref[...], 1.0)
    agg = jnp.zeros((s2_ref.shape[1], 3 * OUT_C), jnp.float32)
    for r in range(NUM_REL):
        w = (att[r, 0] * basis_ref[0]
             + att[r, 1] * basis_ref[1]
             + att[r, 2] * basis_ref[2])
        agg = agg + jnp.dot(s2_ref[r], w,
                            preferred_element_type=jnp.float32)
    g = agg * recip + jnp.dot(x_ref[...], root_ref[...],
                              preferred_element_type=jnp.float32)

    gi = jax.nn.sigmoid(g[:, 0:OUT_C] + bi_ref[...])
    gc = jnp.tanh(g[:, OUT_C:2 * OUT_C] + bc_ref[...])
    go = jax.nn.sigmoid(g[:, 2 * OUT_C:3 * OUT_C] + bo_ref[...])
    next_c = gi * gc
    c_ref[...] = next_c
    h_ref[...] = go * jnp.tanh(next_c)


def kernel(inputs, edge_index, edge_attr, basis_i, att_i, root_i,
           basis_h, att_h, root_h, bias_f, bias_i, bias_c, bias_o):
    del basis_h, att_h, root_h, bias_f
    x = inputs
    x2 = x.reshape(2 * N, HALF)
    ei3 = jnp.concatenate([edge_index.astype(jnp.int32),
                           edge_attr.astype(jnp.int32)[None, :]], axis=0)
    pad = jnp.broadcast_to(
        jnp.array([[0], [DUMMY_DST], [NUM_REL - 1]], jnp.int32),
        (3, E_PAD - E))
    ei3 = jnp.concatenate([ei3, pad], axis=1)

    s2, cnt = _sc_scatter(x2, ei3)

    cnt2 = (cnt[0, :N] + cnt[1, :N]).reshape(N, 1)
    basis2 = basis_i[:, :, OUT_C:]
    root_sl = root_i[:, OUT_C:]
    att_pad = jnp.zeros((8, 128), jnp.float32).at[:NUM_REL, :NUM_REL].set(att_i)
    b_i = bias_i.reshape(1, OUT_C)
    b_c = bias_c.reshape(1, OUT_C)
    b_o = bias_o.reshape(1, OUT_C)

    B = 2000
    grid = (N // B,)
    h, c = pl.pallas_call(
        _tc_body,
        grid=grid,
        in_specs=[
            pl.BlockSpec((NUM_REL, B, IN_C), lambda i: (0, i, 0)),
            pl.BlockSpec((B, 1), lambda i: (i, 0)),
            pl.BlockSpec((B, IN_C), lambda i: (i, 0)),
            pl.BlockSpec((8, 128), lambda i: (0, 0)),
            pl.BlockSpec((NUM_REL, IN_C, 3 * OUT_C), lambda i: (0, 0, 0)),
            pl.BlockSpec((IN_C, 3 * OUT_C), lambda i: (0, 0)),
            pl.BlockSpec((1, OUT_C), lambda i: (0, 0)),
            pl.BlockSpec((1, OUT_C), lambda i: (0, 0)),
            pl.BlockSpec((1, OUT_C), lambda i: (0, 0)),
        ],
        out_specs=[
            pl.BlockSpec((B, OUT_C), lambda i: (i, 0)),
            pl.BlockSpec((B, OUT_C), lambda i: (i, 0)),
        ],
        out_shape=[
            jax.ShapeDtypeStruct((N, OUT_C), jnp.float32),
            jax.ShapeDtypeStruct((N, OUT_C), jnp.float32),
        ],
    )(s2, cnt2, x, att_pad, basis2, root_sl, b_i, b_c, b_o)
    return (h, c)
